# baseline (speedup 1.0000x reference)
import numpy as np
import jax
import jax.numpy as jnp
from jax.experimental import pallas as pl
from jax.experimental.pallas import tpu as pltpu

KSIZE = 7
STRIDE_Q = 4
SCALE = 10.0
TOPK = 100
T, C_IN, H, W = 4, 64, 64, 64
C_INT = 16
NQ_SIDE = 16
NK_SIDE = 64
NQ = NQ_SIDE * NQ_SIDE
NK = NK_SIDE * NK_SIDE
D = C_INT * KSIZE * KSIZE
PT_Q = 1
N_ITERS = 40


def _fold_constants():
    mcols = np.zeros((KSIZE, NQ_SIDE, W), np.float32)
    for dw in range(KSIZE):
        for qj in range(NQ_SIDE):
            s = STRIDE_Q * qj + dw - PT_Q
            if 0 <= s < W:
                mcols[dw, qj, s] = 1.0
    rt = np.zeros((H, KSIZE * NQ_SIDE), np.float32)
    for dh in range(KSIZE):
        for qi in range(NQ_SIDE):
            r = STRIDE_Q * qi + dh - PT_Q
            if 0 <= r < H:
                rt[r, dh * NQ_SIDE + qi] = 1.0
    cov = np.zeros((H,), np.float32)
    for r in range(H):
        rp = r + PT_Q
        for qi in range(NQ_SIDE):
            if 0 <= rp - STRIDE_Q * qi < KSIZE:
                cov[r] += 1.0
    inv_cnt = (1.0 / np.outer(cov, cov)).astype(np.float32)
    return mcols, rt, inv_cnt


_MCOLS, _RT, _INV_CNT = _fold_constants()


def _proj_body(b_ref, w_ref, bias_ref, out_ref):
    out_ref[0] = (
        jnp.dot(w_ref[...], b_ref[0], preferred_element_type=jnp.float32)
        + bias_ref[...]
    )


def _row_reduce(x, f):
    p = f(x.reshape(NQ, NK // 128, 128), axis=1)
    return f(p, axis=1, keepdims=True)


def _wts_body(q_ref, kpat_ref, wts_ref):
    q = q_ref[0]
    kpat = kpat_ref[0]
    d = jax.lax.dot_general(q, kpat, (((1,), (1,)), ((), ())),
                            preferred_element_type=jnp.float32)

    m = _row_reduce(d, jnp.max)
    lo0 = _row_reduce(d, jnp.min)

    def body(_, carry):
        lo, hi = carry
        mid = 0.5 * (lo + hi)
        cnt = _row_reduce((d >= mid).astype(jnp.float32), jnp.sum)
        take = cnt >= TOPK
        return jnp.where(take, mid, lo), jnp.where(take, hi, mid)

    lo, _ = jax.lax.fori_loop(0, N_ITERS, body, (lo0, m))

    e = jnp.where(d >= lo, jnp.exp((d - m) * SCALE), 0.0)
    wts_ref[0] = e / _row_reduce(e, jnp.sum)


def _fold_body(wts_ref, vpat_ref, b_ref, mcols_ref, rt_ref,
               icnt_ref, ww_ref, bw_ref, out_ref):
    z = jnp.dot(wts_ref[0], vpat_ref[0],
                preferred_element_type=jnp.float32)
    zt = z.T

    z5 = zt.reshape(C_INT, KSIZE, KSIZE, NQ_SIDE, NQ_SIDE)
    a = jnp.zeros((C_INT * KSIZE * NQ_SIDE, W), jnp.float32)
    for dw in range(KSIZE):
        s = z5[:, :, dw, :, :].reshape(C_INT * KSIZE * NQ_SIDE, NQ_SIDE)
        a = a + jnp.dot(s, mcols_ref[dw], preferred_element_type=jnp.float32)
    a3 = a.reshape(C_INT, KSIZE * NQ_SIDE, W)
    rt = rt_ref[...]
    icnt = icnt_ref[...]
    ys = [jnp.dot(rt, a3[c], preferred_element_type=jnp.float32) * icnt
          for c in range(C_INT)]
    y = jnp.stack(ys, 0).reshape(C_INT, H * W)

    out_ref[0] = (
        jnp.dot(ww_ref[...], y, preferred_element_type=jnp.float32)
        + bw_ref[...]
        + b_ref[0]
    )


def _unfold(xp, stride):
    n_h = (xp.shape[2] - KSIZE) // stride + 1
    n_w = (xp.shape[3] - KSIZE) // stride + 1
    idx_r = (jnp.arange(n_h) * stride)[:, None] + jnp.arange(KSIZE)[None, :]
    idx_c = (jnp.arange(n_w) * stride)[:, None] + jnp.arange(KSIZE)[None, :]
    p = xp[:, :, idx_r][:, :, :, :, idx_c]
    p = jnp.transpose(p, (0, 2, 4, 1, 3, 5)).reshape(
        xp.shape[0], n_h * n_w, xp.shape[1] * KSIZE * KSIZE)
    return p


def kernel(b, Wg, bg, Wth, bth, Wph, bph, Ww, bw):
    bf = b.reshape(T, C_IN, H * W)
    wcat = jnp.concatenate([Wg, Wth, Wph], axis=0)
    bcat = jnp.concatenate([bg, bth, bph]).reshape(-1, 1)

    proj = pl.pallas_call(
        _proj_body,
        grid=(T,),
        in_specs=[
            pl.BlockSpec((1, C_IN, H * W), lambda t: (t, 0, 0)),
            pl.BlockSpec((3 * C_INT, C_IN), lambda t: (0, 0)),
            pl.BlockSpec((3 * C_INT, 1), lambda t: (0, 0)),
        ],
        out_specs=pl.BlockSpec((1, 3 * C_INT, H * W), lambda t: (t, 0, 0)),
        out_shape=jax.ShapeDtypeStruct((T, 3 * C_INT, H * W), jnp.float32),
    )(bf, wcat, bcat)

    b1 = proj[:, 0:C_INT].reshape(T, C_INT, H, W)
    b2 = proj[:, C_INT:2 * C_INT].reshape(T, C_INT, H, W)
    b3 = proj[:, 2 * C_INT:].reshape(T, C_INT, H, W)

    qp = jnp.pad(b1, ((0, 0), (0, 0), (1, 2), (1, 2)))
    kp = jnp.pad(b3, ((0, 0), (0, 0), (3, 3), (3, 3)))
    vp = jnp.pad(b2, ((0, 0), (0, 0), (3, 3), (3, 3)))

    q = _unfold(qp, STRIDE_Q)
    kpat = _unfold(kp, 1)
    vpat = _unfold(vp, 1)

    wts = pl.pallas_call(
        _wts_body,
        grid=(T,),
        in_specs=[
            pl.BlockSpec((1, NQ, D), lambda t: (t, 0, 0)),
            pl.BlockSpec((1, NK, D), lambda t: (t, 0, 0)),
        ],
        out_specs=pl.BlockSpec((1, NQ, NK), lambda t: (t, 0, 0)),
        out_shape=jax.ShapeDtypeStruct((T, NQ, NK), jnp.float32),
    )(q, kpat)

    out = pl.pallas_call(
        _fold_body,
        grid=(T,),
        in_specs=[
            pl.BlockSpec((1, NQ, NK), lambda t: (t, 0, 0)),
            pl.BlockSpec((1, NK, D), lambda t: (t, 0, 0)),
            pl.BlockSpec((1, C_IN, H * W), lambda t: (t, 0, 0)),
            pl.BlockSpec((KSIZE, NQ_SIDE, W), lambda t: (0, 0, 0)),
            pl.BlockSpec((H, KSIZE * NQ_SIDE), lambda t: (0, 0)),
            pl.BlockSpec((H, W), lambda t: (0, 0)),
            pl.BlockSpec((C_IN, C_INT), lambda t: (0, 0)),
            pl.BlockSpec((C_IN, 1), lambda t: (0, 0)),
        ],
        out_specs=pl.BlockSpec((1, C_IN, H * W), lambda t: (t, 0, 0)),
        out_shape=jax.ShapeDtypeStruct((T, C_IN, H * W), jnp.float32),
    )(wts, vpat, bf, _MCOLS, _RT, _INV_CNT, Ww, bw.reshape(-1, 1))

    return out.reshape(T, C_IN, H, W)


# speedup vs baseline: 1.3199x; 1.3199x over previous
import numpy as np
import jax
import jax.numpy as jnp
from jax.experimental import pallas as pl
from jax.experimental.pallas import tpu as pltpu

KSIZE = 7
STRIDE_Q = 4
SCALE = 10.0
TOPK = 100
T, C_IN, H, W = 4, 64, 64, 64
C_INT = 16
NQ_SIDE = 16
NK_SIDE = 64
NQ = NQ_SIDE * NQ_SIDE
NK = NK_SIDE * NK_SIDE
D = C_INT * KSIZE * KSIZE
PT_Q = 1
N_ITERS = 30


def _fold_constants():
    mcols = np.zeros((KSIZE, NQ_SIDE, W), np.float32)
    for dw in range(KSIZE):
        for qj in range(NQ_SIDE):
            s = STRIDE_Q * qj + dw - PT_Q
            if 0 <= s < W:
                mcols[dw, qj, s] = 1.0
    rt = np.zeros((H, KSIZE * NQ_SIDE), np.float32)
    for dh in range(KSIZE):
        for qi in range(NQ_SIDE):
            r = STRIDE_Q * qi + dh - PT_Q
            if 0 <= r < H:
                rt[r, dh * NQ_SIDE + qi] = 1.0
    cov = np.zeros((H,), np.float32)
    for r in range(H):
        rp = r + PT_Q
        for qi in range(NQ_SIDE):
            if 0 <= rp - STRIDE_Q * qi < KSIZE:
                cov[r] += 1.0
    inv_cnt = (1.0 / np.outer(cov, cov)).astype(np.float32)
    return mcols, rt, inv_cnt


_MCOLS, _RT, _INV_CNT = _fold_constants()


def _proj_body(b_ref, w_ref, bias_ref, out_ref):
    out_ref[0] = (
        jnp.dot(w_ref[...], b_ref[0], preferred_element_type=jnp.float32)
        + bias_ref[...]
    )


def _wts_body(qt_ref, kpat_ref, wts_ref):
    qt = qt_ref[0]
    kpat = kpat_ref[0]
    d = jnp.dot(kpat, qt, preferred_element_type=jnp.float32)

    m = jnp.max(d, axis=0, keepdims=True)
    lo0 = jnp.min(d, axis=0, keepdims=True)

    def body(_, carry):
        lo, hi = carry
        mid = 0.5 * (lo + hi)
        cnt = jnp.sum((d >= mid).astype(jnp.float32), axis=0, keepdims=True)
        take = cnt >= TOPK
        return jnp.where(take, mid, lo), jnp.where(take, hi, mid)

    lo, _ = jax.lax.fori_loop(0, N_ITERS, body, (lo0, m), unroll=3)

    e = jnp.where(d >= lo, jnp.exp((d - m) * SCALE), 0.0)
    wts_ref[0] = e / jnp.sum(e, axis=0, keepdims=True)


def _fold_body(wts_ref, vpat_ref, b_ref, mcols_ref, rt_ref,
               icnt_ref, ww_ref, bw_ref, out_ref):
    zt = jnp.dot(vpat_ref[0], wts_ref[0],
                 preferred_element_type=jnp.float32)

    z5 = zt.reshape(C_INT, KSIZE, KSIZE, NQ_SIDE, NQ_SIDE)
    a = jnp.zeros((C_INT * KSIZE * NQ_SIDE, W), jnp.float32)
    for dw in range(KSIZE):
        s = z5[:, :, dw, :, :].reshape(C_INT * KSIZE * NQ_SIDE, NQ_SIDE)
        a = a + jnp.dot(s, mcols_ref[dw], preferred_element_type=jnp.float32)
    a3 = a.reshape(C_INT, KSIZE * NQ_SIDE, W)
    rt = rt_ref[...]
    icnt = icnt_ref[...]
    ys = [jnp.dot(rt, a3[c], preferred_element_type=jnp.float32) * icnt
          for c in range(C_INT)]
    y = jnp.stack(ys, 0).reshape(C_INT, H * W)

    out_ref[0] = (
        jnp.dot(ww_ref[...], y, preferred_element_type=jnp.float32)
        + bw_ref[...]
        + b_ref[0]
    )


def _unfold(xp, stride):
    n_h = (xp.shape[2] - KSIZE) // stride + 1
    n_w = (xp.shape[3] - KSIZE) // stride + 1
    idx_r = (jnp.arange(n_h) * stride)[:, None] + jnp.arange(KSIZE)[None, :]
    idx_c = (jnp.arange(n_w) * stride)[:, None] + jnp.arange(KSIZE)[None, :]
    p = xp[:, :, idx_r][:, :, :, :, idx_c]
    p = jnp.transpose(p, (0, 2, 4, 1, 3, 5)).reshape(
        xp.shape[0], n_h * n_w, xp.shape[1] * KSIZE * KSIZE)
    return p


def kernel(b, Wg, bg, Wth, bth, Wph, bph, Ww, bw):
    bf = b.reshape(T, C_IN, H * W)
    wcat = jnp.concatenate([Wg, Wth, Wph], axis=0)
    bcat = jnp.concatenate([bg, bth, bph]).reshape(-1, 1)

    proj = pl.pallas_call(
        _proj_body,
        grid=(T,),
        in_specs=[
            pl.BlockSpec((1, C_IN, H * W), lambda t: (t, 0, 0)),
            pl.BlockSpec((3 * C_INT, C_IN), lambda t: (0, 0)),
            pl.BlockSpec((3 * C_INT, 1), lambda t: (0, 0)),
        ],
        out_specs=pl.BlockSpec((1, 3 * C_INT, H * W), lambda t: (t, 0, 0)),
        out_shape=jax.ShapeDtypeStruct((T, 3 * C_INT, H * W), jnp.float32),
    )(bf, wcat, bcat)

    b1 = proj[:, 0:C_INT].reshape(T, C_INT, H, W)
    b2 = proj[:, C_INT:2 * C_INT].reshape(T, C_INT, H, W)
    b3 = proj[:, 2 * C_INT:].reshape(T, C_INT, H, W)

    qp = jnp.pad(b1, ((0, 0), (0, 0), (1, 2), (1, 2)))
    kp = jnp.pad(b3, ((0, 0), (0, 0), (3, 3), (3, 3)))
    vp = jnp.pad(b2, ((0, 0), (0, 0), (3, 3), (3, 3)))

    qt = _unfold(qp, STRIDE_Q).transpose(0, 2, 1)
    kpat = _unfold(kp, 1)
    vpt = _unfold(vp, 1).transpose(0, 2, 1)

    wts = pl.pallas_call(
        _wts_body,
        grid=(T,),
        in_specs=[
            pl.BlockSpec((1, D, NQ), lambda t: (t, 0, 0)),
            pl.BlockSpec((1, NK, D), lambda t: (t, 0, 0)),
        ],
        out_specs=pl.BlockSpec((1, NK, NQ), lambda t: (t, 0, 0)),
        out_shape=jax.ShapeDtypeStruct((T, NK, NQ), jnp.float32),
    )(qt, kpat)

    out = pl.pallas_call(
        _fold_body,
        grid=(T,),
        in_specs=[
            pl.BlockSpec((1, NK, NQ), lambda t: (t, 0, 0)),
            pl.BlockSpec((1, D, NK), lambda t: (t, 0, 0)),
            pl.BlockSpec((1, C_IN, H * W), lambda t: (t, 0, 0)),
            pl.BlockSpec((KSIZE, NQ_SIDE, W), lambda t: (0, 0, 0)),
            pl.BlockSpec((H, KSIZE * NQ_SIDE), lambda t: (0, 0)),
            pl.BlockSpec((H, W), lambda t: (0, 0)),
            pl.BlockSpec((C_IN, C_INT), lambda t: (0, 0)),
            pl.BlockSpec((C_IN, 1), lambda t: (0, 0)),
        ],
        out_specs=pl.BlockSpec((1, C_IN, H * W), lambda t: (t, 0, 0)),
        out_shape=jax.ShapeDtypeStruct((T, C_IN, H * W), jnp.float32),
    )(wts, vpt, bf, _MCOLS, _RT, _INV_CNT, Ww, bw.reshape(-1, 1))

    return out.reshape(T, C_IN, H, W)


# speedup vs baseline: 3.6600x; 2.7730x over previous
import numpy as np
import jax
import jax.numpy as jnp
from jax.experimental import pallas as pl
from jax.experimental.pallas import tpu as pltpu

KSIZE = 7
STRIDE_Q = 4
SCALE = 10.0
TOPK = 100
T, C_IN, H, W = 4, 64, 64, 64
C_INT = 16
NQ_SIDE = 16
NK_SIDE = 64
NQ = NQ_SIDE * NQ_SIDE
NK = NK_SIDE * NK_SIDE
D = C_INT * KSIZE * KSIZE
PT_Q = 1
N_ITERS = 30

KW = NK_SIDE + KSIZE - 1
KPAD = 4992
IMGL = 5504


def _fold_constants():
    mcols = np.zeros((KSIZE, NQ_SIDE, W), np.float32)
    for dw in range(KSIZE):
        for qj in range(NQ_SIDE):
            s = STRIDE_Q * qj + dw - PT_Q
            if 0 <= s < W:
                mcols[dw, qj, s] = 1.0
    rt = np.zeros((H, KSIZE * NQ_SIDE), np.float32)
    for dh in range(KSIZE):
        for qi in range(NQ_SIDE):
            r = STRIDE_Q * qi + dh - PT_Q
            if 0 <= r < H:
                rt[r, dh * NQ_SIDE + qi] = 1.0
    cov = np.zeros((H,), np.float32)
    for r in range(H):
        rp = r + PT_Q
        for qi in range(NQ_SIDE):
            if 0 <= rp - STRIDE_Q * qi < KSIZE:
                cov[r] += 1.0
    inv_cnt = (1.0 / np.outer(cov, cov)).astype(np.float32)
    return mcols, rt, inv_cnt


_MCOLS, _RT, _INV_CNT = _fold_constants()


def _key_mask():
    m = np.zeros((8, KPAD), np.float32)
    for j in range(KW * KW):
        if (j // KW) < NK_SIDE and (j % KW) < NK_SIDE:
            m[0, j] = 1.0
    return m


_KMASK = _key_mask()


def _build_bank(img_ref, bank_s):
    for p in range(KSIZE * KSIZE):
        sh = (p // KSIZE) * KW + (p % KSIZE)
        bank_s[p * C_INT:(p + 1) * C_INT, :] = img_ref[0, :, sh:sh + KPAD]


def _row_red(x, op, fin):
    acc = x[:, 0:128]
    for j in range(1, KPAD // 128):
        acc = op(acc, x[:, j * 128:(j + 1) * 128])
    return fin(acc, axis=1, keepdims=True)


def _proj_body(b_ref, w_ref, bias_ref, out_ref):
    out_ref[0] = (
        jnp.dot(w_ref[...], b_ref[0], preferred_element_type=jnp.float32)
        + bias_ref[...]
    )


def _wts_body(q_ref, kpf_ref, mask_ref, wts_ref, bank_s):
    _build_bank(kpf_ref, bank_s)
    q = q_ref[0]
    d = jnp.dot(q, bank_s[...], preferred_element_type=jnp.float32)
    valid = mask_ref[0:1, :] > 0.5
    dm = jnp.where(valid, d, -3.0e38)

    m = _row_red(dm, jnp.maximum, jnp.max)
    lo0 = _row_red(jnp.where(valid, d, 3.0e38), jnp.minimum, jnp.min)

    def body(_, carry):
        lo, hi = carry
        mid = 0.5 * (lo + hi)
        cnt = _row_red((dm >= mid).astype(jnp.float32), jnp.add, jnp.sum)
        take = cnt >= TOPK
        return jnp.where(take, mid, lo), jnp.where(take, hi, mid)

    lo, _ = jax.lax.fori_loop(0, N_ITERS, body, (lo0, m), unroll=3)

    e = jnp.where(dm >= lo, jnp.exp((dm - m) * SCALE), 0.0)
    wts_ref[0] = e / _row_red(e, jnp.add, jnp.sum)


def _fold_body(wts_ref, vpf_ref, b_ref, mcols_ref, rt_ref,
               icnt_ref, ww_ref, bw_ref, out_ref, bank_s):
    _build_bank(vpf_ref, bank_s)
    zt = jax.lax.dot_general(bank_s[...], wts_ref[0],
                             (((1,), (1,)), ((), ())),
                             preferred_element_type=jnp.float32)

    z5 = zt.reshape(KSIZE, KSIZE, C_INT, NQ_SIDE, NQ_SIDE)
    a = jnp.zeros((KSIZE * C_INT * NQ_SIDE, W), jnp.float32)
    for dw in range(KSIZE):
        s = z5[:, dw].reshape(KSIZE * C_INT * NQ_SIDE, NQ_SIDE)
        a = a + jnp.dot(s, mcols_ref[dw], preferred_element_type=jnp.float32)
    a3 = a.reshape(KSIZE, C_INT, NQ_SIDE, W).transpose(1, 0, 2, 3)
    a3 = a3.reshape(C_INT, KSIZE * NQ_SIDE, W)
    rt = rt_ref[...]
    icnt = icnt_ref[...]
    ys = [jnp.dot(rt, a3[c], preferred_element_type=jnp.float32) * icnt
          for c in range(C_INT)]
    y = jnp.stack(ys, 0).reshape(C_INT, H * W)

    out_ref[0] = (
        jnp.dot(ww_ref[...], y, preferred_element_type=jnp.float32)
        + bw_ref[...]
        + b_ref[0]
    )


def _unfold(xp, stride):
    n_h = (xp.shape[2] - KSIZE) // stride + 1
    n_w = (xp.shape[3] - KSIZE) // stride + 1
    idx_r = (jnp.arange(n_h) * stride)[:, None] + jnp.arange(KSIZE)[None, :]
    idx_c = (jnp.arange(n_w) * stride)[:, None] + jnp.arange(KSIZE)[None, :]
    p = xp[:, :, idx_r][:, :, :, :, idx_c]
    p = jnp.transpose(p, (0, 2, 4, 1, 3, 5)).reshape(
        xp.shape[0], n_h * n_w, xp.shape[1] * KSIZE * KSIZE)
    return p


def kernel(b, Wg, bg, Wth, bth, Wph, bph, Ww, bw):
    bf = b.reshape(T, C_IN, H * W)
    wcat = jnp.concatenate([Wg, Wth, Wph], axis=0)
    bcat = jnp.concatenate([bg, bth, bph]).reshape(-1, 1)

    proj = pl.pallas_call(
        _proj_body,
        grid=(T,),
        in_specs=[
            pl.BlockSpec((1, C_IN, H * W), lambda t: (t, 0, 0)),
            pl.BlockSpec((3 * C_INT, C_IN), lambda t: (0, 0)),
            pl.BlockSpec((3 * C_INT, 1), lambda t: (0, 0)),
        ],
        out_specs=pl.BlockSpec((1, 3 * C_INT, H * W), lambda t: (t, 0, 0)),
        out_shape=jax.ShapeDtypeStruct((T, 3 * C_INT, H * W), jnp.float32),
    )(bf, wcat, bcat)

    b1 = proj[:, 0:C_INT].reshape(T, C_INT, H, W)
    b2 = proj[:, C_INT:2 * C_INT].reshape(T, C_INT, H, W)
    b3 = proj[:, 2 * C_INT:].reshape(T, C_INT, H, W)

    qp = jnp.pad(b1, ((0, 0), (0, 0), (1, 2), (1, 2)))
    kp = jnp.pad(b3, ((0, 0), (0, 0), (3, 3), (3, 3)))
    vp = jnp.pad(b2, ((0, 0), (0, 0), (3, 3), (3, 3)))

    q = _unfold(qp, STRIDE_Q)
    q_r = q.reshape(T, NQ, C_INT, KSIZE * KSIZE)
    q_r = q_r.transpose(0, 1, 3, 2).reshape(T, NQ, D)
    kpf = jnp.pad(kp.reshape(T, C_INT, KW * KW),
                  ((0, 0), (0, 0), (0, IMGL - KW * KW)))
    vpf = jnp.pad(vp.reshape(T, C_INT, KW * KW),
                  ((0, 0), (0, 0), (0, IMGL - KW * KW)))

    wts = pl.pallas_call(
        _wts_body,
        grid=(T,),
        in_specs=[
            pl.BlockSpec((1, NQ, D), lambda t: (t, 0, 0)),
            pl.BlockSpec((1, C_INT, IMGL), lambda t: (t, 0, 0)),
            pl.BlockSpec((8, KPAD), lambda t: (0, 0)),
        ],
        out_specs=pl.BlockSpec((1, NQ, KPAD), lambda t: (t, 0, 0)),
        out_shape=jax.ShapeDtypeStruct((T, NQ, KPAD), jnp.float32),
        scratch_shapes=[pltpu.VMEM((D, KPAD), jnp.float32)],
    )(q_r, kpf, _KMASK)

    out = pl.pallas_call(
        _fold_body,
        grid=(T,),
        in_specs=[
            pl.BlockSpec((1, NQ, KPAD), lambda t: (t, 0, 0)),
            pl.BlockSpec((1, C_INT, IMGL), lambda t: (t, 0, 0)),
            pl.BlockSpec((1, C_IN, H * W), lambda t: (t, 0, 0)),
            pl.BlockSpec((KSIZE, NQ_SIDE, W), lambda t: (0, 0, 0)),
            pl.BlockSpec((H, KSIZE * NQ_SIDE), lambda t: (0, 0)),
            pl.BlockSpec((H, W), lambda t: (0, 0)),
            pl.BlockSpec((C_IN, C_INT), lambda t: (0, 0)),
            pl.BlockSpec((C_IN, 1), lambda t: (0, 0)),
        ],
        out_specs=pl.BlockSpec((1, C_IN, H * W), lambda t: (t, 0, 0)),
        out_shape=jax.ShapeDtypeStruct((T, C_IN, H * W), jnp.float32),
        scratch_shapes=[pltpu.VMEM((D, KPAD), jnp.float32)],
    )(wts, vpf, bf, _MCOLS, _RT, _INV_CNT, Ww, bw.reshape(-1, 1))

    return out.reshape(T, C_IN, H, W)


# speedup vs baseline: 3.6901x; 1.0082x over previous
import numpy as np
import jax
import jax.numpy as jnp
from jax.experimental import pallas as pl
from jax.experimental.pallas import tpu as pltpu

KSIZE = 7
STRIDE_Q = 4
SCALE = 10.0
TOPK = 100
T, C_IN, H, W = 4, 64, 64, 64
C_INT = 16
NQ_SIDE = 16
NK_SIDE = 64
NQ = NQ_SIDE * NQ_SIDE
NK = NK_SIDE * NK_SIDE
D = C_INT * KSIZE * KSIZE
PT_Q = 1
N_ITERS = 30

KW = NK_SIDE + KSIZE - 1
KPAD = 4992
IMGL = 5504


def _fold_constants():
    mcols = np.zeros((KSIZE, NQ_SIDE, W), np.float32)
    for dw in range(KSIZE):
        for qj in range(NQ_SIDE):
            s = STRIDE_Q * qj + dw - PT_Q
            if 0 <= s < W:
                mcols[dw, qj, s] = 1.0
    rt = np.zeros((H, KSIZE * NQ_SIDE), np.float32)
    for dh in range(KSIZE):
        for qi in range(NQ_SIDE):
            r = STRIDE_Q * qi + dh - PT_Q
            if 0 <= r < H:
                rt[r, dh * NQ_SIDE + qi] = 1.0
    cov = np.zeros((H,), np.float32)
    for r in range(H):
        rp = r + PT_Q
        for qi in range(NQ_SIDE):
            if 0 <= rp - STRIDE_Q * qi < KSIZE:
                cov[r] += 1.0
    inv_cnt = (1.0 / np.outer(cov, cov)).astype(np.float32)
    return mcols, rt, inv_cnt


_MCOLS, _RT, _INV_CNT = _fold_constants()


def _key_mask():
    m = np.zeros((8, KPAD), np.float32)
    for j in range(KW * KW):
        if (j // KW) < NK_SIDE and (j % KW) < NK_SIDE:
            m[0, j] = 1.0
    return m


_KMASK = _key_mask()


def _build_bank(img_ref, bank_s):
    for p in range(KSIZE * KSIZE):
        sh = (p // KSIZE) * KW + (p % KSIZE)
        bank_s[p * C_INT:(p + 1) * C_INT, :] = img_ref[0, :, sh:sh + KPAD]


def _row_red(x, op, fin):
    acc = x[:, 0:128]
    for j in range(1, KPAD // 128):
        acc = op(acc, x[:, j * 128:(j + 1) * 128])
    return fin(acc, axis=1, keepdims=True)


def _proj_body(b_ref, w_ref, bias_ref, out_ref):
    out_ref[0] = (
        jnp.dot(w_ref[...], b_ref[0], preferred_element_type=jnp.float32)
        + bias_ref[...]
    )


def _attn_body(q_ref, kpf_ref, vpf_ref, mask_ref, b_ref, mcols_ref, rt_ref,
               icnt_ref, ww_ref, bw_ref, out_ref, bank_s):
    _build_bank(kpf_ref, bank_s)
    q = q_ref[0]
    d = jnp.dot(q, bank_s[...], preferred_element_type=jnp.float32)
    valid = mask_ref[0:1, :] > 0.5
    dm = jnp.where(valid, d, -3.0e38)

    m = _row_red(dm, jnp.maximum, jnp.max)
    lo0 = _row_red(jnp.where(valid, d, 3.0e38), jnp.minimum, jnp.min)

    def body(_, carry):
        lo, hi = carry
        mid = 0.5 * (lo + hi)
        cnt = _row_red((dm >= mid).astype(jnp.float32), jnp.add, jnp.sum)
        take = cnt >= TOPK
        return jnp.where(take, mid, lo), jnp.where(take, hi, mid)

    lo, _ = jax.lax.fori_loop(0, N_ITERS, body, (lo0, m), unroll=3)

    e = jnp.where(dm >= lo, jnp.exp((dm - m) * SCALE), 0.0)
    wts = e / _row_red(e, jnp.add, jnp.sum)

    _build_bank(vpf_ref, bank_s)
    zt = jax.lax.dot_general(bank_s[...], wts,
                             (((1,), (1,)), ((), ())),
                             preferred_element_type=jnp.float32)

    z5 = zt.reshape(KSIZE, KSIZE, C_INT, NQ_SIDE, NQ_SIDE)
    a = jnp.zeros((KSIZE * C_INT * NQ_SIDE, W), jnp.float32)
    for dw in range(KSIZE):
        s = z5[:, dw].reshape(KSIZE * C_INT * NQ_SIDE, NQ_SIDE)
        a = a + jnp.dot(s, mcols_ref[dw], preferred_element_type=jnp.float32)
    a3 = a.reshape(KSIZE, C_INT, NQ_SIDE, W).transpose(1, 0, 2, 3)
    a3 = a3.reshape(C_INT, KSIZE * NQ_SIDE, W)
    rt = rt_ref[...]
    icnt = icnt_ref[...]
    ys = [jnp.dot(rt, a3[c], preferred_element_type=jnp.float32) * icnt
          for c in range(C_INT)]
    y = jnp.stack(ys, 0).reshape(C_INT, H * W)

    out_ref[0] = (
        jnp.dot(ww_ref[...], y, preferred_element_type=jnp.float32)
        + bw_ref[...]
        + b_ref[0]
    )


def _unfold(xp, stride):
    n_h = (xp.shape[2] - KSIZE) // stride + 1
    n_w = (xp.shape[3] - KSIZE) // stride + 1
    idx_r = (jnp.arange(n_h) * stride)[:, None] + jnp.arange(KSIZE)[None, :]
    idx_c = (jnp.arange(n_w) * stride)[:, None] + jnp.arange(KSIZE)[None, :]
    p = xp[:, :, idx_r][:, :, :, :, idx_c]
    p = jnp.transpose(p, (0, 2, 4, 1, 3, 5)).reshape(
        xp.shape[0], n_h * n_w, xp.shape[1] * KSIZE * KSIZE)
    return p


def kernel(b, Wg, bg, Wth, bth, Wph, bph, Ww, bw):
    bf = b.reshape(T, C_IN, H * W)
    wcat = jnp.concatenate([Wg, Wth, Wph], axis=0)
    bcat = jnp.concatenate([bg, bth, bph]).reshape(-1, 1)

    proj = pl.pallas_call(
        _proj_body,
        grid=(T,),
        in_specs=[
            pl.BlockSpec((1, C_IN, H * W), lambda t: (t, 0, 0)),
            pl.BlockSpec((3 * C_INT, C_IN), lambda t: (0, 0)),
            pl.BlockSpec((3 * C_INT, 1), lambda t: (0, 0)),
        ],
        out_specs=pl.BlockSpec((1, 3 * C_INT, H * W), lambda t: (t, 0, 0)),
        out_shape=jax.ShapeDtypeStruct((T, 3 * C_INT, H * W), jnp.float32),
    )(bf, wcat, bcat)

    b1 = proj[:, 0:C_INT].reshape(T, C_INT, H, W)
    b2 = proj[:, C_INT:2 * C_INT].reshape(T, C_INT, H, W)
    b3 = proj[:, 2 * C_INT:].reshape(T, C_INT, H, W)

    qp = jnp.pad(b1, ((0, 0), (0, 0), (1, 2), (1, 2)))
    kp = jnp.pad(b3, ((0, 0), (0, 0), (3, 3), (3, 3)))
    vp = jnp.pad(b2, ((0, 0), (0, 0), (3, 3), (3, 3)))

    q = _unfold(qp, STRIDE_Q)
    q_r = q.reshape(T, NQ, C_INT, KSIZE * KSIZE)
    q_r = q_r.transpose(0, 1, 3, 2).reshape(T, NQ, D)
    kpf = jnp.pad(kp.reshape(T, C_INT, KW * KW),
                  ((0, 0), (0, 0), (0, IMGL - KW * KW)))
    vpf = jnp.pad(vp.reshape(T, C_INT, KW * KW),
                  ((0, 0), (0, 0), (0, IMGL - KW * KW)))

    out = pl.pallas_call(
        _attn_body,
        grid=(T,),
        in_specs=[
            pl.BlockSpec((1, NQ, D), lambda t: (t, 0, 0)),
            pl.BlockSpec((1, C_INT, IMGL), lambda t: (t, 0, 0)),
            pl.BlockSpec((1, C_INT, IMGL), lambda t: (t, 0, 0)),
            pl.BlockSpec((8, KPAD), lambda t: (0, 0)),
            pl.BlockSpec((1, C_IN, H * W), lambda t: (t, 0, 0)),
            pl.BlockSpec((KSIZE, NQ_SIDE, W), lambda t: (0, 0, 0)),
            pl.BlockSpec((H, KSIZE * NQ_SIDE), lambda t: (0, 0)),
            pl.BlockSpec((H, W), lambda t: (0, 0)),
            pl.BlockSpec((C_IN, C_INT), lambda t: (0, 0)),
            pl.BlockSpec((C_IN, 1), lambda t: (0, 0)),
        ],
        out_specs=pl.BlockSpec((1, C_IN, H * W), lambda t: (t, 0, 0)),
        out_shape=jax.ShapeDtypeStruct((T, C_IN, H * W), jnp.float32),
        scratch_shapes=[pltpu.VMEM((D, KPAD), jnp.float32)],
    )(q_r, kpf, vpf, _KMASK, bf, _MCOLS, _RT, _INV_CNT, Ww,
      bw.reshape(-1, 1))

    return out.reshape(T, C_IN, H, W)


# speedup vs baseline: 3.8025x; 1.0305x over previous
import numpy as np
import jax
import jax.numpy as jnp
from jax.experimental import pallas as pl
from jax.experimental.pallas import tpu as pltpu

KSIZE = 7
STRIDE_Q = 4
SCALE = 10.0
TOPK = 100
T, C_IN, H, W = 4, 64, 64, 64
C_INT = 16
NQ_SIDE = 16
NK_SIDE = 64
NQ = NQ_SIDE * NQ_SIDE
NK = NK_SIDE * NK_SIDE
D = C_INT * KSIZE * KSIZE
PT_Q = 1
N_ITERS = 27

KW = NK_SIDE + KSIZE - 1
KPAD = 4992
IMGL = 5504


def _fold_constants():
    mcols = np.zeros((KSIZE, NQ_SIDE, W), np.float32)
    for dw in range(KSIZE):
        for qj in range(NQ_SIDE):
            s = STRIDE_Q * qj + dw - PT_Q
            if 0 <= s < W:
                mcols[dw, qj, s] = 1.0
    rt = np.zeros((H, KSIZE * NQ_SIDE), np.float32)
    for dh in range(KSIZE):
        for qi in range(NQ_SIDE):
            r = STRIDE_Q * qi + dh - PT_Q
            if 0 <= r < H:
                rt[r, dh * NQ_SIDE + qi] = 1.0
    cov = np.zeros((H,), np.float32)
    for r in range(H):
        rp = r + PT_Q
        for qi in range(NQ_SIDE):
            if 0 <= rp - STRIDE_Q * qi < KSIZE:
                cov[r] += 1.0
    inv_cnt = (1.0 / np.outer(cov, cov)).astype(np.float32)
    return mcols, rt, inv_cnt


_MCOLS, _RT, _INV_CNT = _fold_constants()


def _key_mask():
    m = np.zeros((8, KPAD), np.float32)
    for j in range(KW * KW):
        if (j // KW) < NK_SIDE and (j % KW) < NK_SIDE:
            m[0, j] = 1.0
    return m


_KMASK = _key_mask()


def _build_bank(img_ref, bank_s):
    for p in range(KSIZE * KSIZE):
        sh = (p // KSIZE) * KW + (p % KSIZE)
        bank_s[p * C_INT:(p + 1) * C_INT, :] = img_ref[0, :, sh:sh + KPAD]


def _row_red(x, op, fin):
    parts = [x[:, j * 128:(j + 1) * 128] for j in range(KPAD // 128)]
    while len(parts) > 1:
        nxt = [op(parts[i], parts[i + 1])
               for i in range(0, len(parts) - 1, 2)]
        if len(parts) % 2:
            nxt.append(parts[-1])
        parts = nxt
    return fin(parts[0], axis=1, keepdims=True)


def _proj_body(b_ref, w_ref, bias_ref, out_ref):
    out_ref[0] = (
        jnp.dot(w_ref[...], b_ref[0], preferred_element_type=jnp.float32)
        + bias_ref[...]
    )


def _attn_body(q_ref, kpf_ref, vpf_ref, mask_ref, b_ref, mcols_ref, rt_ref,
               icnt_ref, ww_ref, bw_ref, out_ref, bank_s):
    _build_bank(kpf_ref, bank_s)
    q = q_ref[0]
    d = jnp.dot(q, bank_s[...], preferred_element_type=jnp.float32)
    valid = mask_ref[0:1, :] > 0.5
    dm = jnp.where(valid, d, -3.0e38)

    m = _row_red(dm, jnp.maximum, jnp.max)
    lo0 = _row_red(jnp.where(valid, d, 3.0e38), jnp.minimum, jnp.min)

    def body(_, carry):
        lo, hi = carry
        mid = 0.5 * (lo + hi)
        cnt = _row_red((dm >= mid).astype(jnp.float32), jnp.add, jnp.sum)
        take = cnt >= TOPK
        return jnp.where(take, mid, lo), jnp.where(take, hi, mid)

    lo, _ = jax.lax.fori_loop(0, N_ITERS, body, (lo0, m), unroll=3)

    e = jnp.where(dm >= lo, jnp.exp((dm - m) * SCALE), 0.0)
    wts = e / _row_red(e, jnp.add, jnp.sum)

    _build_bank(vpf_ref, bank_s)
    zt = jax.lax.dot_general(bank_s[...], wts,
                             (((1,), (1,)), ((), ())),
                             preferred_element_type=jnp.float32)

    z5 = zt.reshape(KSIZE, KSIZE, C_INT, NQ_SIDE, NQ_SIDE)
    a = jnp.zeros((KSIZE * C_INT * NQ_SIDE, W), jnp.float32)
    for dw in range(KSIZE):
        s = z5[:, dw].reshape(KSIZE * C_INT * NQ_SIDE, NQ_SIDE)
        a = a + jnp.dot(s, mcols_ref[dw], preferred_element_type=jnp.float32)
    a3 = a.reshape(KSIZE, C_INT, NQ_SIDE, W).transpose(1, 0, 2, 3)
    a3 = a3.reshape(C_INT, KSIZE * NQ_SIDE, W)
    rt = rt_ref[...]
    icnt = icnt_ref[...]
    ys = [jnp.dot(rt, a3[c], preferred_element_type=jnp.float32) * icnt
          for c in range(C_INT)]
    y = jnp.stack(ys, 0).reshape(C_INT, H * W)

    out_ref[0] = (
        jnp.dot(ww_ref[...], y, preferred_element_type=jnp.float32)
        + bw_ref[...]
        + b_ref[0]
    )


def _unfold(xp, stride):
    n_h = (xp.shape[2] - KSIZE) // stride + 1
    n_w = (xp.shape[3] - KSIZE) // stride + 1
    idx_r = (jnp.arange(n_h) * stride)[:, None] + jnp.arange(KSIZE)[None, :]
    idx_c = (jnp.arange(n_w) * stride)[:, None] + jnp.arange(KSIZE)[None, :]
    p = xp[:, :, idx_r][:, :, :, :, idx_c]
    p = jnp.transpose(p, (0, 2, 4, 1, 3, 5)).reshape(
        xp.shape[0], n_h * n_w, xp.shape[1] * KSIZE * KSIZE)
    return p


def kernel(b, Wg, bg, Wth, bth, Wph, bph, Ww, bw):
    bf = b.reshape(T, C_IN, H * W)
    wcat = jnp.concatenate([Wg, Wth, Wph], axis=0)
    bcat = jnp.concatenate([bg, bth, bph]).reshape(-1, 1)

    proj = pl.pallas_call(
        _proj_body,
        grid=(T,),
        in_specs=[
            pl.BlockSpec((1, C_IN, H * W), lambda t: (t, 0, 0)),
            pl.BlockSpec((3 * C_INT, C_IN), lambda t: (0, 0)),
            pl.BlockSpec((3 * C_INT, 1), lambda t: (0, 0)),
        ],
        out_specs=pl.BlockSpec((1, 3 * C_INT, H * W), lambda t: (t, 0, 0)),
        out_shape=jax.ShapeDtypeStruct((T, 3 * C_INT, H * W), jnp.float32),
    )(bf, wcat, bcat)

    b1 = proj[:, 0:C_INT].reshape(T, C_INT, H, W)
    b2 = proj[:, C_INT:2 * C_INT].reshape(T, C_INT, H, W)
    b3 = proj[:, 2 * C_INT:].reshape(T, C_INT, H, W)

    qp = jnp.pad(b1, ((0, 0), (0, 0), (1, 2), (1, 2)))
    kp = jnp.pad(b3, ((0, 0), (0, 0), (3, 3), (3, 3)))
    vp = jnp.pad(b2, ((0, 0), (0, 0), (3, 3), (3, 3)))

    q = _unfold(qp, STRIDE_Q)
    q_r = q.reshape(T, NQ, C_INT, KSIZE * KSIZE)
    q_r = q_r.transpose(0, 1, 3, 2).reshape(T, NQ, D)
    kpf = jnp.pad(kp.reshape(T, C_INT, KW * KW),
                  ((0, 0), (0, 0), (0, IMGL - KW * KW)))
    vpf = jnp.pad(vp.reshape(T, C_INT, KW * KW),
                  ((0, 0), (0, 0), (0, IMGL - KW * KW)))

    out = pl.pallas_call(
        _attn_body,
        grid=(T,),
        in_specs=[
            pl.BlockSpec((1, NQ, D), lambda t: (t, 0, 0)),
            pl.BlockSpec((1, C_INT, IMGL), lambda t: (t, 0, 0)),
            pl.BlockSpec((1, C_INT, IMGL), lambda t: (t, 0, 0)),
            pl.BlockSpec((8, KPAD), lambda t: (0, 0)),
            pl.BlockSpec((1, C_IN, H * W), lambda t: (t, 0, 0)),
            pl.BlockSpec((KSIZE, NQ_SIDE, W), lambda t: (0, 0, 0)),
            pl.BlockSpec((H, KSIZE * NQ_SIDE), lambda t: (0, 0)),
            pl.BlockSpec((H, W), lambda t: (0, 0)),
            pl.BlockSpec((C_IN, C_INT), lambda t: (0, 0)),
            pl.BlockSpec((C_IN, 1), lambda t: (0, 0)),
        ],
        out_specs=pl.BlockSpec((1, C_IN, H * W), lambda t: (t, 0, 0)),
        out_shape=jax.ShapeDtypeStruct((T, C_IN, H * W), jnp.float32),
        scratch_shapes=[pltpu.VMEM((D, KPAD), jnp.float32)],
    )(q_r, kpf, vpf, _KMASK, bf, _MCOLS, _RT, _INV_CNT, Ww,
      bw.reshape(-1, 1))

    return out.reshape(T, C_IN, H, W)


# speedup vs baseline: 3.9234x; 1.0318x over previous
import numpy as np
import jax
import jax.numpy as jnp
from jax.experimental import pallas as pl
from jax.experimental.pallas import tpu as pltpu

KSIZE = 7
STRIDE_Q = 4
SCALE = 10.0
TOPK = 100
T, C_IN, H, W = 4, 64, 64, 64
C_INT = 16
NQ_SIDE = 16
NK_SIDE = 64
NQ = NQ_SIDE * NQ_SIDE
NK = NK_SIDE * NK_SIDE
D = C_INT * KSIZE * KSIZE
PT_Q = 1
N_ITERS = 27

KW = NK_SIDE + KSIZE - 1
KPAD = 4992
IMGL = 5504


def _fold_constants():
    mcols = np.zeros((KSIZE, NQ_SIDE, W), np.float32)
    for dw in range(KSIZE):
        for qj in range(NQ_SIDE):
            s = STRIDE_Q * qj + dw - PT_Q
            if 0 <= s < W:
                mcols[dw, qj, s] = 1.0
    rt = np.zeros((H, KSIZE * NQ_SIDE), np.float32)
    for dh in range(KSIZE):
        for qi in range(NQ_SIDE):
            r = STRIDE_Q * qi + dh - PT_Q
            if 0 <= r < H:
                rt[r, dh * NQ_SIDE + qi] = 1.0
    cov = np.zeros((H,), np.float32)
    for r in range(H):
        rp = r + PT_Q
        for qi in range(NQ_SIDE):
            if 0 <= rp - STRIDE_Q * qi < KSIZE:
                cov[r] += 1.0
    inv_cnt = (1.0 / np.outer(cov, cov)).astype(np.float32)
    return mcols, rt, inv_cnt


_MCOLS, _RT, _INV_CNT = _fold_constants()


def _key_mask():
    m = np.zeros((8, KPAD), np.float32)
    for j in range(KW * KW):
        if (j // KW) < NK_SIDE and (j % KW) < NK_SIDE:
            m[0, j] = 1.0
    return m


_KMASK = _key_mask()


def _fill_img(img_s, feat):
    img_s[...] = jnp.zeros((C_INT, IMGL), jnp.float32)
    for r in range(H):
        base = (r + 3) * KW + 3
        img_s[:, base:base + W] = feat[:, r * W:(r + 1) * W]


def _build_bank(img_s, bank_s):
    for p in range(KSIZE * KSIZE):
        sh = (p // KSIZE) * KW + (p % KSIZE)
        bank_s[p * C_INT:(p + 1) * C_INT, :] = img_s[:, sh:sh + KPAD]


def _row_red(x, op, fin):
    parts = [x[:, j * 128:(j + 1) * 128] for j in range(KPAD // 128)]
    while len(parts) > 1:
        nxt = [op(parts[i], parts[i + 1])
               for i in range(0, len(parts) - 1, 2)]
        if len(parts) % 2:
            nxt.append(parts[-1])
        parts = nxt
    return fin(parts[0], axis=1, keepdims=True)


def _proj_body(b_ref, w_ref, bias_ref, out_ref):
    out_ref[0] = (
        jnp.dot(w_ref[...], b_ref[0], preferred_element_type=jnp.float32)
        + bias_ref[...]
    )


def _attn_body(q_ref, wkv_ref, bkv_ref, mask_ref, b_ref, mcols_ref, rt_ref,
               icnt_ref, ww_ref, bw_ref, out_ref, bank_s, img_s):
    kv = (jnp.dot(wkv_ref[...], b_ref[0],
                  preferred_element_type=jnp.float32) + bkv_ref[...])
    _fill_img(img_s, kv[C_INT:2 * C_INT, :])
    _build_bank(img_s, bank_s)
    q = q_ref[0]
    d = jnp.dot(q, bank_s[...], preferred_element_type=jnp.float32)
    valid = mask_ref[0:1, :] > 0.5
    dm = jnp.where(valid, d, -3.0e38)

    m = _row_red(dm, jnp.maximum, jnp.max)
    lo0 = _row_red(jnp.where(valid, d, 3.0e38), jnp.minimum, jnp.min)

    def body(_, carry):
        lo, hi = carry
        mid = 0.5 * (lo + hi)
        cnt = _row_red((dm >= mid).astype(jnp.float32), jnp.add, jnp.sum)
        take = cnt >= TOPK
        return jnp.where(take, mid, lo), jnp.where(take, hi, mid)

    lo, _ = jax.lax.fori_loop(0, N_ITERS, body, (lo0, m), unroll=3)

    e = jnp.where(dm >= lo, jnp.exp((dm - m) * SCALE), 0.0)
    wts = e / _row_red(e, jnp.add, jnp.sum)

    _fill_img(img_s, kv[0:C_INT, :])
    _build_bank(img_s, bank_s)
    zt = jax.lax.dot_general(bank_s[...], wts,
                             (((1,), (1,)), ((), ())),
                             preferred_element_type=jnp.float32)

    z5 = zt.reshape(KSIZE, KSIZE, C_INT, NQ_SIDE, NQ_SIDE)
    a = jnp.zeros((KSIZE * C_INT * NQ_SIDE, W), jnp.float32)
    for dw in range(KSIZE):
        s = z5[:, dw].reshape(KSIZE * C_INT * NQ_SIDE, NQ_SIDE)
        a = a + jnp.dot(s, mcols_ref[dw], preferred_element_type=jnp.float32)
    a3 = a.reshape(KSIZE, C_INT, NQ_SIDE, W).transpose(1, 0, 2, 3)
    a3 = a3.reshape(C_INT, KSIZE * NQ_SIDE, W)
    rt = rt_ref[...]
    icnt = icnt_ref[...]
    ys = [jnp.dot(rt, a3[c], preferred_element_type=jnp.float32) * icnt
          for c in range(C_INT)]
    y = jnp.stack(ys, 0).reshape(C_INT, H * W)

    out_ref[0] = (
        jnp.dot(ww_ref[...], y, preferred_element_type=jnp.float32)
        + bw_ref[...]
        + b_ref[0]
    )


def _unfold(xp, stride):
    n_h = (xp.shape[2] - KSIZE) // stride + 1
    n_w = (xp.shape[3] - KSIZE) // stride + 1
    idx_r = (jnp.arange(n_h) * stride)[:, None] + jnp.arange(KSIZE)[None, :]
    idx_c = (jnp.arange(n_w) * stride)[:, None] + jnp.arange(KSIZE)[None, :]
    p = xp[:, :, idx_r][:, :, :, :, idx_c]
    p = jnp.transpose(p, (0, 2, 4, 1, 3, 5)).reshape(
        xp.shape[0], n_h * n_w, xp.shape[1] * KSIZE * KSIZE)
    return p


def kernel(b, Wg, bg, Wth, bth, Wph, bph, Ww, bw):
    bf = b.reshape(T, C_IN, H * W)
    wkv = jnp.concatenate([Wth, Wph], axis=0)
    bkv = jnp.concatenate([bth, bph]).reshape(-1, 1)

    b1f = pl.pallas_call(
        _proj_body,
        grid=(T,),
        in_specs=[
            pl.BlockSpec((1, C_IN, H * W), lambda t: (t, 0, 0)),
            pl.BlockSpec((C_INT, C_IN), lambda t: (0, 0)),
            pl.BlockSpec((C_INT, 1), lambda t: (0, 0)),
        ],
        out_specs=pl.BlockSpec((1, C_INT, H * W), lambda t: (t, 0, 0)),
        out_shape=jax.ShapeDtypeStruct((T, C_INT, H * W), jnp.float32),
    )(bf, Wg, bg.reshape(-1, 1))

    qp = jnp.pad(b1f.reshape(T, C_INT, H, W),
                 ((0, 0), (0, 0), (1, 2), (1, 2)))
    q = _unfold(qp, STRIDE_Q)
    q_r = q.reshape(T, NQ, C_INT, KSIZE * KSIZE)
    q_r = q_r.transpose(0, 1, 3, 2).reshape(T, NQ, D)

    out = pl.pallas_call(
        _attn_body,
        grid=(T,),
        in_specs=[
            pl.BlockSpec((1, NQ, D), lambda t: (t, 0, 0)),
            pl.BlockSpec((2 * C_INT, C_IN), lambda t: (0, 0)),
            pl.BlockSpec((2 * C_INT, 1), lambda t: (0, 0)),
            pl.BlockSpec((8, KPAD), lambda t: (0, 0)),
            pl.BlockSpec((1, C_IN, H * W), lambda t: (t, 0, 0)),
            pl.BlockSpec((KSIZE, NQ_SIDE, W), lambda t: (0, 0, 0)),
            pl.BlockSpec((H, KSIZE * NQ_SIDE), lambda t: (0, 0)),
            pl.BlockSpec((H, W), lambda t: (0, 0)),
            pl.BlockSpec((C_IN, C_INT), lambda t: (0, 0)),
            pl.BlockSpec((C_IN, 1), lambda t: (0, 0)),
        ],
        out_specs=pl.BlockSpec((1, C_IN, H * W), lambda t: (t, 0, 0)),
        out_shape=jax.ShapeDtypeStruct((T, C_IN, H * W), jnp.float32),
        scratch_shapes=[pltpu.VMEM((D, KPAD), jnp.float32),
                        pltpu.VMEM((C_INT, IMGL), jnp.float32)],
    )(q_r, wkv, bkv, _KMASK, bf, _MCOLS, _RT, _INV_CNT, Ww,
      bw.reshape(-1, 1))

    return out.reshape(T, C_IN, H, W)
